# baseline (speedup 1.0000x reference)
import functools

import jax
import jax.numpy as jnp
from jax import lax
from jax.experimental import pallas as pl
from jax.experimental.pallas import tpu as pltpu
from jax.experimental.pallas import tpu_sc as plsc

N = 10000
E = 160000
L = 128
STEPS = 4

NC = 2
NS = 16
NW = NC * NS

CH = 128
EP = 163840
NCHUNK = EP // CH
NITER = NCHUNK // NW
NP = N + 8


@functools.lru_cache(maxsize=None)
def _sc_mesh():
    return plsc.VectorSubcoreMesh(
        core_axis_name="c", subcore_axis_name="s",
        num_cores=NC, num_subcores=NS)


def _gather2_body(h_hbm, src_hbm, dst_hbm, hs_out, hd_out,
                  sidx, didx, srows, drows, sem1, sem2):
    cid = lax.axis_index("c")
    sid = lax.axis_index("s")
    wid = sid * NC + cid

    def body(k, carry):
        chunk = wid + k * NW

        @pl.when(chunk < NCHUNK)
        def _():
            off = chunk * CH
            pltpu.sync_copy(src_hbm.at[pl.ds(off, CH)], sidx)
            pltpu.sync_copy(dst_hbm.at[pl.ds(off, CH)], didx)
            c1 = pltpu.async_copy(h_hbm.at[sidx], srows, sem1)
            c2 = pltpu.async_copy(h_hbm.at[didx], drows, sem2)
            c1.wait()
            c2.wait()
            pltpu.sync_copy(srows, hs_out.at[pl.ds(off, CH)])
            pltpu.sync_copy(drows, hd_out.at[pl.ds(off, CH)])

        return carry

    lax.fori_loop(0, NITER, body, 0)


def _gather2(h_node, src, dst):
    f = pl.kernel(
        _gather2_body,
        out_type=(jax.ShapeDtypeStruct((EP, L), jnp.float32),
                  jax.ShapeDtypeStruct((EP, L), jnp.float32)),
        mesh=_sc_mesh(),
        scratch_types=[
            pltpu.VMEM((CH,), jnp.int32),
            pltpu.VMEM((CH,), jnp.int32),
            pltpu.VMEM((CH, L), jnp.float32),
            pltpu.VMEM((CH, L), jnp.float32),
            pltpu.SemaphoreType.DMA,
            pltpu.SemaphoreType.DMA,
        ],
    )
    return f(h_node, src, dst)


_ROWS_PER_TILE = 624
_TAIL_ROWS = NP - NS * _ROWS_PER_TILE


def _scatter_body(m_hbm, dst_hbm, zeros_hbm, out_hbm, accum, idxb, rows):
    cid = lax.axis_index("c")
    sid = lax.axis_index("s")
    wid = sid * NC + cid

    r0 = sid * _ROWS_PER_TILE
    pltpu.sync_copy(zeros_hbm.at[pl.ds(r0, _ROWS_PER_TILE)],
                    accum.at[pl.ds(r0, _ROWS_PER_TILE)])

    @pl.when(sid == 0)
    def _():
        pltpu.sync_copy(zeros_hbm.at[pl.ds(NS * _ROWS_PER_TILE, _TAIL_ROWS)],
                        accum.at[pl.ds(NS * _ROWS_PER_TILE, _TAIL_ROWS)])

    plsc.subcore_barrier()

    def body(k, carry):
        chunk = wid + k * NW

        @pl.when(chunk < NCHUNK)
        def _():
            off = chunk * CH
            pltpu.sync_copy(dst_hbm.at[pl.ds(off, CH)], idxb)
            pltpu.sync_copy(m_hbm.at[pl.ds(off, CH)], rows)
            pltpu.sync_copy(rows, accum.at[idxb], add=True)

        return carry

    lax.fori_loop(0, NITER, body, 0)
    plsc.subcore_barrier()
    pltpu.sync_copy(accum.at[pl.ds(r0, _ROWS_PER_TILE)],
                    out_hbm.at[cid, pl.ds(r0, _ROWS_PER_TILE)])

    @pl.when(sid == 0)
    def _():
        pltpu.sync_copy(accum.at[pl.ds(NS * _ROWS_PER_TILE, _TAIL_ROWS)],
                        out_hbm.at[cid, pl.ds(NS * _ROWS_PER_TILE, _TAIL_ROWS)])


def _scatter_add(m, dst, zeros_nl):
    f = pl.kernel(
        _scatter_body,
        out_type=jax.ShapeDtypeStruct((NC, NP, L), jnp.float32),
        mesh=_sc_mesh(),
        scratch_types=[
            pltpu.VMEM_SHARED((NP, L), jnp.float32),
            pltpu.VMEM((CH,), jnp.int32),
            pltpu.VMEM((CH, L), jnp.float32),
        ],
    )
    return f(m, dst, zeros_nl)


def _ln(h, g, beta):
    mu = jnp.mean(h, axis=-1, keepdims=True)
    var = jnp.mean((h - mu) * (h - mu), axis=-1, keepdims=True)
    return (h - mu) * lax.rsqrt(var + 1e-5) * g + beta


def _dot(a, b):
    return jnp.dot(a, b, preferred_element_type=jnp.float32)


def _node_encoder_body(x_ref, w1, b1, w2, b2, g, beta, out_ref):
    h = jnp.maximum(_dot(x_ref[...], w1[...]) + b1[...], 0.0)
    h = jnp.maximum(_dot(h, w2[...]) + b2[...], 0.0)
    out_ref[...] = _ln(h, g[...], beta[...])


def _edge_encoder_body(ea_ref, w1, b1, w2, b2, g, beta, out_ref):
    h = jnp.maximum(ea_ref[...] * w1[...] + b1[...], 0.0)
    h = jnp.maximum(_dot(h, w2[...]) + b2[...], 0.0)
    out_ref[...] = _ln(h, g[...], beta[...])


def _edge_step_body(hd_ref, hs_ref, he_ref, wa, wb, wc, b1, w2, b2, g, beta,
                    m_ref, heo_ref):
    hd = hd_ref[...]
    hs = hs_ref[...]
    he = he_ref[...]
    c = _dot(he, wc[...]) + b1[...]
    pa = _dot(hd, wa[...])
    pb = _dot(hs, wb[...])
    qa = _dot(hs, wa[...])
    qb = _dot(hd, wb[...])

    hm = jnp.maximum(pa + pb + c, 0.0)
    hm = jnp.maximum(_dot(hm, w2[...]) + b2[...], 0.0)
    m_ref[...] = _ln(hm, g[...], beta[...])

    hx = jnp.maximum(qa + qb + c, 0.0)
    hx = jnp.maximum(_dot(hx, w2[...]) + b2[...], 0.0)
    heo_ref[...] = _ln(hx, g[...], beta[...]) + he


def _node_step_body(p_ref, h_ref, wa, wb, b1, w2, b2, g, beta, out_ref):
    h = h_ref[...]
    aggr = p_ref[0] + p_ref[1]
    u = jnp.maximum(_dot(aggr, wa[...]) + _dot(h, wb[...]) + b1[...], 0.0)
    u = jnp.maximum(_dot(u, w2[...]) + b2[...], 0.0)
    out_ref[...] = _ln(u, g[...], beta[...]) + h


def _decoder_body(h_ref, w1, b1, w2, b2, out_ref):
    u = jnp.maximum(_dot(h_ref[...], w1[...]) + b1[...], 0.0)
    out_ref[...] = _dot(u, w2[...]) + b2[...]


def _row(v):
    return v.reshape(1, -1)


BE = 2000
BN = 2000


def _wspec(shape):
    return pl.BlockSpec(shape, lambda i: tuple(0 for _ in shape))


def kernel(mean_stress, pos, nodes_types, edge_attr, edge_index, params):
    x = jnp.hstack([mean_stress, pos, nodes_types])
    x = jnp.pad(x, ((0, 0), (0, 1)))
    src = jnp.pad(edge_index[0], (0, EP - E))
    dst = jnp.pad(edge_index[1], (0, EP - E))
    dstS = jnp.pad(edge_index[1], (0, EP - E), constant_values=N)
    edge_attr_p = jnp.pad(edge_attr, (0, EP - E))

    ne, ee, pe, pn, dec = (params["ne"], params["ee"], params["pe"],
                           params["pn"], params["dec"])

    w1n = jnp.pad(ne["W1"], ((0, 1), (0, 0)))

    h_node = pl.pallas_call(
        _node_encoder_body,
        out_shape=jax.ShapeDtypeStruct((N, L), jnp.float32),
        grid=(1,),
        in_specs=[_wspec((N, 8)), _wspec((8, L)), _wspec((1, L)),
                  _wspec((L, L)), _wspec((1, L)), _wspec((1, L)),
                  _wspec((1, L))],
        out_specs=_wspec((N, L)),
    )(x, w1n, _row(ne["b1"]), ne["W2"], _row(ne["b2"]), _row(ne["g"]),
      _row(ne["beta"]))

    h_edge = pl.pallas_call(
        _edge_encoder_body,
        out_shape=jax.ShapeDtypeStruct((EP, L), jnp.float32),
        grid=(EP // BE,),
        in_specs=[pl.BlockSpec((BE, 1), lambda i: (i, 0)),
                  _wspec((1, L)), _wspec((1, L)), _wspec((L, L)),
                  _wspec((1, L)), _wspec((1, L)), _wspec((1, L))],
        out_specs=pl.BlockSpec((BE, L), lambda i: (i, 0)),
    )(edge_attr_p.reshape(EP, 1), ee["W1"], _row(ee["b1"]), ee["W2"],
      _row(ee["b2"]), _row(ee["g"]), _row(ee["beta"]))

    wa = pe["W1"][:L]
    wb = pe["W1"][L:2 * L]
    wc = pe["W1"][2 * L:]
    wna = pn["W1"][:L]
    wnb = pn["W1"][L:]

    zeros_nl = jnp.zeros((NP, L), jnp.float32)

    edge_step = pl.pallas_call(
        _edge_step_body,
        out_shape=(jax.ShapeDtypeStruct((EP, L), jnp.float32),
                   jax.ShapeDtypeStruct((EP, L), jnp.float32)),
        grid=(EP // BE,),
        in_specs=[pl.BlockSpec((BE, L), lambda i: (i, 0)),
                  pl.BlockSpec((BE, L), lambda i: (i, 0)),
                  pl.BlockSpec((BE, L), lambda i: (i, 0)),
                  _wspec((L, L)), _wspec((L, L)), _wspec((L, L)),
                  _wspec((1, L)), _wspec((L, L)), _wspec((1, L)),
                  _wspec((1, L)), _wspec((1, L))],
        out_specs=(pl.BlockSpec((BE, L), lambda i: (i, 0)),
                   pl.BlockSpec((BE, L), lambda i: (i, 0))),
    )

    node_step = pl.pallas_call(
        _node_step_body,
        out_shape=jax.ShapeDtypeStruct((N, L), jnp.float32),
        grid=(N // BN,),
        in_specs=[pl.BlockSpec((NC, BN, L), lambda i: (0, i, 0)),
                  pl.BlockSpec((BN, L), lambda i: (i, 0)),
                  _wspec((L, L)), _wspec((L, L)), _wspec((1, L)),
                  _wspec((L, L)), _wspec((1, L)), _wspec((1, L)),
                  _wspec((1, L))],
        out_specs=pl.BlockSpec((BN, L), lambda i: (i, 0)),
    )

    for _ in range(STEPS):
        hs, hd = _gather2(h_node, src, dst)
        m, h_edge = edge_step(hd, hs, h_edge, wa, wb, wc, _row(pe["b1"]),
                              pe["W2"], _row(pe["b2"]), _row(pe["g"]),
                              _row(pe["beta"]))
        partials = _scatter_add(m, dstS, zeros_nl)
        h_node = node_step(partials, h_node, wna, wnb, _row(pn["b1"]),
                           pn["W2"], _row(pn["b2"]), _row(pn["g"]),
                           _row(pn["beta"]))

    w2d = jnp.pad(dec["W2"], ((0, 0), (0, 5)))
    b2d = jnp.pad(dec["b2"], (0, 5))
    decoded = pl.pallas_call(
        _decoder_body,
        out_shape=jax.ShapeDtypeStruct((N, 8), jnp.float32),
        grid=(1,),
        in_specs=[_wspec((N, L)), _wspec((L, L)), _wspec((1, L)),
                  _wspec((L, 8)), _wspec((1, 8))],
        out_specs=_wspec((N, 8)),
    )(h_node, dec["W1"], _row(dec["b1"]), w2d, _row(b2d))

    return decoded[:, :3]


# speedup vs baseline: 1.6238x; 1.6238x over previous
import functools

import jax
import jax.numpy as jnp
from jax import lax
from jax.experimental import pallas as pl
from jax.experimental.pallas import tpu as pltpu
from jax.experimental.pallas import tpu_sc as plsc

N = 10000
E = 160000
L = 128
STEPS = 4

NC = 2
NS = 16
NW = NC * NS

CH = 128
NCHUNK = E // CH
NITER = -(-NCHUNK // NW)


@functools.lru_cache(maxsize=None)
def _sc_mesh():
    return plsc.VectorSubcoreMesh(
        core_axis_name="c", subcore_axis_name="s",
        num_cores=NC, num_subcores=NS)


def _gather2_body(h_hbm, src_hbm, dst_hbm, hs_out, hd_out,
                  sidx, didx, srows, drows, sem1, sem2):
    cid = lax.axis_index("c")
    sid = lax.axis_index("s")
    wid = sid * NC + cid

    def body(k, carry):
        chunk = wid + k * NW

        @pl.when(chunk < NCHUNK)
        def _():
            off = chunk * CH
            pltpu.sync_copy(src_hbm.at[pl.ds(off, CH)], sidx)
            pltpu.sync_copy(dst_hbm.at[pl.ds(off, CH)], didx)
            c1 = pltpu.async_copy(h_hbm.at[sidx], srows, sem1)
            c2 = pltpu.async_copy(h_hbm.at[didx], drows, sem2)
            c1.wait()
            c2.wait()
            pltpu.sync_copy(srows, hs_out.at[pl.ds(off, CH)])
            pltpu.sync_copy(drows, hd_out.at[pl.ds(off, CH)])

        return carry

    lax.fori_loop(0, NITER, body, 0)


def _gather2(h_node, src, dst):
    f = pl.kernel(
        _gather2_body,
        out_type=(jax.ShapeDtypeStruct((E, L), jnp.float32),
                  jax.ShapeDtypeStruct((E, L), jnp.float32)),
        mesh=_sc_mesh(),
        scratch_types=[
            pltpu.VMEM((CH,), jnp.int32),
            pltpu.VMEM((CH,), jnp.int32),
            pltpu.VMEM((CH, L), jnp.float32),
            pltpu.VMEM((CH, L), jnp.float32),
            pltpu.SemaphoreType.DMA,
            pltpu.SemaphoreType.DMA,
        ],
    )
    return f(h_node, src, dst)


_ROWS_PER_TILE = 624
_TAIL_ROWS = N - NS * _ROWS_PER_TILE


def _scatter_body(m_hbm, dst_hbm, zeros_hbm, out_hbm, accum, idxb, rows):
    cid = lax.axis_index("c")
    sid = lax.axis_index("s")
    wid = sid * NC + cid

    r0 = sid * _ROWS_PER_TILE
    pltpu.sync_copy(zeros_hbm.at[pl.ds(r0, _ROWS_PER_TILE)],
                    accum.at[pl.ds(r0, _ROWS_PER_TILE)])

    @pl.when(sid == 0)
    def _():
        pltpu.sync_copy(zeros_hbm.at[pl.ds(NS * _ROWS_PER_TILE, _TAIL_ROWS)],
                        accum.at[pl.ds(NS * _ROWS_PER_TILE, _TAIL_ROWS)])

    plsc.subcore_barrier()

    def body(k, carry):
        chunk = wid + k * NW

        @pl.when(chunk < NCHUNK)
        def _():
            off = chunk * CH
            pltpu.sync_copy(dst_hbm.at[pl.ds(off, CH)], idxb)
            pltpu.sync_copy(m_hbm.at[pl.ds(off, CH)], rows)
            pltpu.sync_copy(rows, accum.at[idxb], add=True)

        return carry

    lax.fori_loop(0, NITER, body, 0)
    plsc.subcore_barrier()
    pltpu.sync_copy(accum.at[pl.ds(r0, _ROWS_PER_TILE)],
                    out_hbm.at[cid, pl.ds(r0, _ROWS_PER_TILE)])

    @pl.when(sid == 0)
    def _():
        pltpu.sync_copy(accum.at[pl.ds(NS * _ROWS_PER_TILE, _TAIL_ROWS)],
                        out_hbm.at[cid, pl.ds(NS * _ROWS_PER_TILE, _TAIL_ROWS)])


def _scatter_add(m, dst, zeros_nl):
    f = pl.kernel(
        _scatter_body,
        out_type=jax.ShapeDtypeStruct((NC, N, L), jnp.float32),
        mesh=_sc_mesh(),
        scratch_types=[
            pltpu.VMEM_SHARED((N, L), jnp.float32),
            pltpu.VMEM((CH,), jnp.int32),
            pltpu.VMEM((CH, L), jnp.float32),
        ],
    )
    return f(m, dst, zeros_nl)


def _ln(h, g, beta):
    mu = jnp.mean(h, axis=-1, keepdims=True)
    var = jnp.mean((h - mu) * (h - mu), axis=-1, keepdims=True)
    return (h - mu) * lax.rsqrt(var + 1e-5) * g + beta


def _dot(a, b):
    return jnp.dot(a, b, preferred_element_type=jnp.float32)


def _node_encoder_body(x_ref, w1, b1, w2, b2, g, beta, out_ref):
    h = jnp.maximum(_dot(x_ref[...], w1[...]) + b1[...], 0.0)
    h = jnp.maximum(_dot(h, w2[...]) + b2[...], 0.0)
    out_ref[...] = _ln(h, g[...], beta[...])


def _edge_encoder_body(ea_ref, w1, b1, w2, b2, g, beta, out_ref):
    h = jnp.maximum(ea_ref[...] * w1[...] + b1[...], 0.0)
    h = jnp.maximum(_dot(h, w2[...]) + b2[...], 0.0)
    out_ref[...] = _ln(h, g[...], beta[...])


def _edge_step_body(hd_ref, hs_ref, he_ref, wa, wb, wc, b1, w2, b2, g, beta,
                    m_ref, heo_ref):
    hd = hd_ref[...]
    hs = hs_ref[...]
    he = he_ref[...]
    c = _dot(he, wc[...]) + b1[...]
    pa = _dot(hd, wa[...])
    pb = _dot(hs, wb[...])
    qa = _dot(hs, wa[...])
    qb = _dot(hd, wb[...])

    hm = jnp.maximum(pa + pb + c, 0.0)
    hm = jnp.maximum(_dot(hm, w2[...]) + b2[...], 0.0)
    m_ref[...] = _ln(hm, g[...], beta[...])

    hx = jnp.maximum(qa + qb + c, 0.0)
    hx = jnp.maximum(_dot(hx, w2[...]) + b2[...], 0.0)
    heo_ref[...] = _ln(hx, g[...], beta[...]) + he


def _node_step_body(p_ref, h_ref, wa, wb, b1, w2, b2, g, beta, out_ref):
    h = h_ref[...]
    aggr = p_ref[0] + p_ref[1]
    u = jnp.maximum(_dot(aggr, wa[...]) + _dot(h, wb[...]) + b1[...], 0.0)
    u = jnp.maximum(_dot(u, w2[...]) + b2[...], 0.0)
    out_ref[...] = _ln(u, g[...], beta[...]) + h


def _decoder_body(h_ref, w1, b1, w2, b2, out_ref):
    u = jnp.maximum(_dot(h_ref[...], w1[...]) + b1[...], 0.0)
    out_ref[...] = _dot(u, w2[...]) + b2[...]


def _row(v):
    return v.reshape(1, -1)


BE = 2000
BN = 2000


def _wspec(shape):
    return pl.BlockSpec(shape, lambda i: tuple(0 for _ in shape))


def kernel(mean_stress, pos, nodes_types, edge_attr, edge_index, params):
    x = jnp.hstack([mean_stress, pos, nodes_types])
    x = jnp.pad(x, ((0, 0), (0, 1)))
    src = edge_index[0]
    dst = edge_index[1]

    ne, ee, pe, pn, dec = (params["ne"], params["ee"], params["pe"],
                           params["pn"], params["dec"])

    w1n = jnp.pad(ne["W1"], ((0, 1), (0, 0)))

    h_node = pl.pallas_call(
        _node_encoder_body,
        out_shape=jax.ShapeDtypeStruct((N, L), jnp.float32),
        grid=(1,),
        in_specs=[_wspec((N, 8)), _wspec((8, L)), _wspec((1, L)),
                  _wspec((L, L)), _wspec((1, L)), _wspec((1, L)),
                  _wspec((1, L))],
        out_specs=_wspec((N, L)),
    )(x, w1n, _row(ne["b1"]), ne["W2"], _row(ne["b2"]), _row(ne["g"]),
      _row(ne["beta"]))

    h_edge = pl.pallas_call(
        _edge_encoder_body,
        out_shape=jax.ShapeDtypeStruct((E, L), jnp.float32),
        grid=(E // BE,),
        in_specs=[pl.BlockSpec((BE, 1), lambda i: (i, 0)),
                  _wspec((1, L)), _wspec((1, L)), _wspec((L, L)),
                  _wspec((1, L)), _wspec((1, L)), _wspec((1, L))],
        out_specs=pl.BlockSpec((BE, L), lambda i: (i, 0)),
    )(edge_attr.reshape(E, 1), ee["W1"], _row(ee["b1"]), ee["W2"],
      _row(ee["b2"]), _row(ee["g"]), _row(ee["beta"]))

    wa = pe["W1"][:L]
    wb = pe["W1"][L:2 * L]
    wc = pe["W1"][2 * L:]
    wna = pn["W1"][:L]
    wnb = pn["W1"][L:]

    zeros_nl = jnp.zeros((N, L), jnp.float32)

    edge_step = pl.pallas_call(
        _edge_step_body,
        out_shape=(jax.ShapeDtypeStruct((E, L), jnp.float32),
                   jax.ShapeDtypeStruct((E, L), jnp.float32)),
        grid=(E // BE,),
        in_specs=[pl.BlockSpec((BE, L), lambda i: (i, 0)),
                  pl.BlockSpec((BE, L), lambda i: (i, 0)),
                  pl.BlockSpec((BE, L), lambda i: (i, 0)),
                  _wspec((L, L)), _wspec((L, L)), _wspec((L, L)),
                  _wspec((1, L)), _wspec((L, L)), _wspec((1, L)),
                  _wspec((1, L)), _wspec((1, L))],
        out_specs=(pl.BlockSpec((BE, L), lambda i: (i, 0)),
                   pl.BlockSpec((BE, L), lambda i: (i, 0))),
    )

    node_step = pl.pallas_call(
        _node_step_body,
        out_shape=jax.ShapeDtypeStruct((N, L), jnp.float32),
        grid=(N // BN,),
        in_specs=[pl.BlockSpec((NC, BN, L), lambda i: (0, i, 0)),
                  pl.BlockSpec((BN, L), lambda i: (i, 0)),
                  _wspec((L, L)), _wspec((L, L)), _wspec((1, L)),
                  _wspec((L, L)), _wspec((1, L)), _wspec((1, L)),
                  _wspec((1, L))],
        out_specs=pl.BlockSpec((BN, L), lambda i: (i, 0)),
    )

    for _ in range(STEPS):
        hs, hd = _gather2(h_node, src, dst)
        m, h_edge = edge_step(hd, hs, h_edge, wa, wb, wc, _row(pe["b1"]),
                              pe["W2"], _row(pe["b2"]), _row(pe["g"]),
                              _row(pe["beta"]))
        partials = _scatter_add(m, dst, zeros_nl)
        h_node = node_step(partials, h_node, wna, wnb, _row(pn["b1"]),
                           pn["W2"], _row(pn["b2"]), _row(pn["g"]),
                           _row(pn["beta"]))

    w2d = jnp.pad(dec["W2"], ((0, 0), (0, 5)))
    b2d = jnp.pad(dec["b2"], (0, 5))
    decoded = pl.pallas_call(
        _decoder_body,
        out_shape=jax.ShapeDtypeStruct((N, 8), jnp.float32),
        grid=(1,),
        in_specs=[_wspec((N, L)), _wspec((L, L)), _wspec((1, L)),
                  _wspec((L, 8)), _wspec((1, 8))],
        out_specs=_wspec((N, 8)),
    )(h_node, dec["W1"], _row(dec["b1"]), w2d, _row(b2d))

    return decoded[:, :3]
